# Initial kernel scaffold; baseline (speedup 1.0000x reference)
import functools

import jax
import jax.numpy as jnp
from jax import lax
from jax.experimental import pallas as pl
from jax.experimental.pallas import tpu as pltpu
from jax.experimental.pallas import tpu_sc as plsc

_EMBED = 64
_BATCH = 16384
_N_NEG = 5
_NW = 32
_ROWS = _BATCH // _NW


def _sc_gather(pos_w, pos_v, neg_t, w_emb, v_emb):
    mesh = plsc.VectorSubcoreMesh(core_axis_name="c", subcore_axis_name="s")

    @functools.partial(
        pl.kernel,
        out_type=(
            jax.ShapeDtypeStruct((_BATCH, _EMBED), jnp.float32),
            jax.ShapeDtypeStruct((_BATCH, _EMBED), jnp.float32),
            jax.ShapeDtypeStruct((_N_NEG * _BATCH, _EMBED), jnp.float32),
        ),
        mesh=mesh,
        scratch_types=[
            pltpu.VMEM((_ROWS,), jnp.int32),
            pltpu.VMEM((_ROWS, _EMBED), jnp.float32),
            pltpu.SemaphoreType.DMA,
        ],
    )
    def k(pos_w_hbm, pos_v_hbm, neg_hbm, w_hbm, v_hbm,
          gw_hbm, gv_hbm, gneg_hbm, idx_v, rows_v, sem):
        wid = lax.axis_index("s") * 2 + lax.axis_index("c")
        base = wid * _ROWS

        pltpu.sync_copy(pos_w_hbm.at[pl.ds(base, _ROWS)], idx_v)
        pltpu.async_copy(w_hbm.at[idx_v], rows_v, sem).wait()
        pltpu.sync_copy(rows_v, gw_hbm.at[pl.ds(base, _ROWS)])

        pltpu.sync_copy(pos_v_hbm.at[pl.ds(base, _ROWS)], idx_v)
        pltpu.async_copy(v_hbm.at[idx_v], rows_v, sem).wait()
        pltpu.sync_copy(rows_v, gv_hbm.at[pl.ds(base, _ROWS)])

        for n in range(_N_NEG):
            off = n * _BATCH + base
            pltpu.sync_copy(neg_hbm.at[pl.ds(off, _ROWS)], idx_v)
            pltpu.async_copy(v_hbm.at[idx_v], rows_v, sem).wait()
            pltpu.sync_copy(rows_v, gneg_hbm.at[pl.ds(off, _ROWS)])

    return k(pos_w, pos_v, neg_t, w_emb, v_emb)


def _log_sigmoid(x):
    return jnp.minimum(x, 0.0) - jnp.log(1.0 + jnp.exp(-jnp.abs(x)))


_BB = 2048


def _tc_loss_body(gw_ref, gv_ref, gneg_ref, out_ref):
    i = pl.program_id(0)
    w = gw_ref[...]
    v = gv_ref[...]
    s = jnp.sum(w * v, axis=1)
    s = jnp.clip(s, -10.0, 10.0)
    part = jnp.sum(_log_sigmoid(s))
    neg = gneg_ref[...]
    ns = jnp.sum(neg * w[None, :, :], axis=2)
    ns = jnp.clip(ns, -10.0, 10.0)
    part = part + jnp.sum(_log_sigmoid(-ns))

    @pl.when(i == 0)
    def _():
        out_ref[0, 0] = 0.0

    out_ref[0, 0] += -part


def _tc_loss(gw, gv, gneg3):
    out = pl.pallas_call(
        _tc_loss_body,
        grid=(_BATCH // _BB,),
        in_specs=[
            pl.BlockSpec((_BB, _EMBED), lambda i: (i, 0)),
            pl.BlockSpec((_BB, _EMBED), lambda i: (i, 0)),
            pl.BlockSpec((_N_NEG, _BB, _EMBED), lambda i: (0, i, 0)),
        ],
        out_specs=pl.BlockSpec(memory_space=pltpu.SMEM),
        out_shape=jax.ShapeDtypeStruct((1, 1), jnp.float32),
    )(gw, gv, gneg3)
    return out[0, 0]


def kernel(pos_w, pos_v, neg_v, w_emb, v_emb):
    neg_t = neg_v.astype(jnp.int32).T.reshape(_N_NEG * _BATCH)
    gw, gv, gneg = _sc_gather(pos_w.astype(jnp.int32),
                              pos_v.astype(jnp.int32),
                              neg_t, w_emb, v_emb)
    gneg3 = gneg.reshape(_N_NEG, _BATCH, _EMBED)
    return _tc_loss(gw, gv, gneg3)


# speedup vs baseline: 1.6429x; 1.6429x over previous
import functools

import jax
import jax.numpy as jnp
from jax import lax
from jax.experimental import pallas as pl
from jax.experimental.pallas import tpu as pltpu
from jax.experimental.pallas import tpu_sc as plsc

_EMBED = 64
_BATCH = 16384
_N_NEG = 5
_NW = 32
_ROWS = _BATCH // _NW


def _sc_gather(pos_w, pos_v, neg_t, w_emb, v_emb):
    mesh = plsc.VectorSubcoreMesh(core_axis_name="c", subcore_axis_name="s")

    @functools.partial(
        pl.kernel,
        out_type=(
            jax.ShapeDtypeStruct((_BATCH, _EMBED), jnp.float32),
            jax.ShapeDtypeStruct((_BATCH, _EMBED), jnp.float32),
            jax.ShapeDtypeStruct((_N_NEG * _BATCH, _EMBED), jnp.float32),
        ),
        mesh=mesh,
        scratch_types=[
            pltpu.VMEM((_ROWS,), jnp.int32),
            pltpu.VMEM((_ROWS, _EMBED), jnp.float32),
            pltpu.SemaphoreType.DMA,
        ],
        compiler_params=pltpu.CompilerParams(use_tc_tiling_on_sc=False),
    )
    def k(pos_w_hbm, pos_v_hbm, neg_hbm, w_hbm, v_hbm,
          gw_hbm, gv_hbm, gneg_hbm, idx_v, rows_v, sem):
        wid = lax.axis_index("s") * 2 + lax.axis_index("c")
        base = wid * _ROWS

        pltpu.sync_copy(pos_w_hbm.at[pl.ds(base, _ROWS)], idx_v)
        pltpu.async_copy(w_hbm.at[idx_v], rows_v, sem).wait()
        pltpu.sync_copy(rows_v, gw_hbm.at[pl.ds(base, _ROWS)])

        pltpu.sync_copy(pos_v_hbm.at[pl.ds(base, _ROWS)], idx_v)
        pltpu.async_copy(v_hbm.at[idx_v], rows_v, sem).wait()
        pltpu.sync_copy(rows_v, gv_hbm.at[pl.ds(base, _ROWS)])

        for n in range(_N_NEG):
            off = n * _BATCH + base
            pltpu.sync_copy(neg_hbm.at[pl.ds(off, _ROWS)], idx_v)
            pltpu.async_copy(v_hbm.at[idx_v], rows_v, sem).wait()
            pltpu.sync_copy(rows_v, gneg_hbm.at[pl.ds(off, _ROWS)])

    return k(pos_w, pos_v, neg_t, w_emb, v_emb)


def _log_sigmoid(x):
    return jnp.minimum(x, 0.0) - jnp.log(1.0 + jnp.exp(-jnp.abs(x)))


_BB = 2048


def _tc_loss_body(gw_ref, gv_ref, gneg_ref, out_ref):
    i = pl.program_id(0)
    w = gw_ref[...]
    v = gv_ref[...]
    s = jnp.sum(w * v, axis=1)
    s = jnp.clip(s, -10.0, 10.0)
    part = jnp.sum(_log_sigmoid(s))
    neg = gneg_ref[...]
    ns = jnp.sum(neg * w[None, :, :], axis=2)
    ns = jnp.clip(ns, -10.0, 10.0)
    part = part + jnp.sum(_log_sigmoid(-ns))

    @pl.when(i == 0)
    def _():
        out_ref[0, 0] = 0.0

    out_ref[0, 0] += -part


def _tc_loss(gw, gv, gneg3):
    out = pl.pallas_call(
        _tc_loss_body,
        grid=(_BATCH // _BB,),
        in_specs=[
            pl.BlockSpec((_BB, _EMBED), lambda i: (i, 0)),
            pl.BlockSpec((_BB, _EMBED), lambda i: (i, 0)),
            pl.BlockSpec((_N_NEG, _BB, _EMBED), lambda i: (0, i, 0)),
        ],
        out_specs=pl.BlockSpec(memory_space=pltpu.SMEM),
        out_shape=jax.ShapeDtypeStruct((1, 1), jnp.float32),
    )(gw, gv, gneg3)
    return out[0, 0]


def kernel(pos_w, pos_v, neg_v, w_emb, v_emb):
    neg_t = neg_v.astype(jnp.int32).T.reshape(_N_NEG * _BATCH)
    gw, gv, gneg = _sc_gather(pos_w.astype(jnp.int32),
                              pos_v.astype(jnp.int32),
                              neg_t, w_emb, v_emb)
    gneg3 = gneg.reshape(_N_NEG, _BATCH, _EMBED)
    return _tc_loss(gw, gv, gneg3)
